# baseline (speedup 1.0000x reference)
import functools

import jax
import jax.numpy as jnp
from jax.experimental import pallas as pl
from jax.experimental.pallas import tpu as pltpu

_EPS = 1e-5
_VMEM_LIMIT = 48 * 1024 * 1024


def _lane_shift(x, d):
    S = x.shape[-1]
    d = d % S
    if d == 0:
        return x
    return jnp.concatenate([x[:, d:], x[:, :d]], axis=-1)


def _gather3x3(x, H, W):
    S = H * W
    col = jax.lax.broadcasted_iota(jnp.int32, (1, S), 1)
    yy = col // W
    xx = col % W
    parts = []
    for wy in range(3):
        for wx in range(3):
            dy, dx = wy - 1, wx - 1
            shifted = _lane_shift(x, dy * W + dx)
            valid = ((yy + dy >= 0) & (yy + dy < H) &
                     (xx + dx >= 0) & (xx + dx < W))
            parts.append(jnp.where(valid, shifted, 0.0))
    return jnp.concatenate(parts, axis=0)


def _gen_noise(x, noise, upper, lower):
    S = x.shape[-1]
    cmax = jnp.max(x, axis=-1, keepdims=True)
    s = jnp.sum(x, axis=-1, keepdims=True)
    q = jnp.sum(x * x, axis=-1, keepdims=True)
    mean = s * (1.0 / S)
    var = jnp.maximum((q - S * mean * mean) * (1.0 / (S - 1)), 0.0)
    std = jnp.sqrt(var)
    clone = jnp.where(x < -cmax * (1.0 / lower), 0.0, x)
    clone = jnp.where(clone > cmax * (1.0 / upper), 0.0, clone)
    return x + clone * (noise * std)


def _conv3x3(a, wmat, bias, H, W):
    patches = _gather3x3(a.astype(jnp.bfloat16), H, W)
    y = jnp.dot(wmat, patches, preferred_element_type=jnp.float32)
    return y + bias


def _fold_stats(st, C):
    Rp = st.shape[0]
    ri = jax.lax.broadcasted_iota(jnp.int32, (C, Rp), 0)
    cj = jax.lax.broadcasted_iota(jnp.int32, (C, Rp), 1) // 4
    fold = (ri == cj).astype(jnp.float32)
    return jnp.dot(fold, st, preferred_element_type=jnp.float32)


def _bn_apply(x_f32, st_sum, g, b, cnt, eps):
    mean = st_sum[:, 0:1] * (1.0 / cnt)
    var = jnp.maximum(st_sum[:, 1:2] * (1.0 / cnt) - mean * mean, 0.0)
    scale = g * jax.lax.rsqrt(var + eps)
    shift = b - mean * scale
    return jnp.maximum(x_f32 * scale + shift, 0.0)


def _stage1_kernel(a_ref, w_ref, g_ref, b_ref, o_ref, *, eps):
    y = jnp.dot(a_ref[...], w_ref[...], preferred_element_type=jnp.float32)
    m = y.shape[0]
    mean = jnp.sum(y, axis=0, keepdims=True) * (1.0 / m)
    var = jnp.maximum(jnp.sum(y * y, axis=0, keepdims=True) * (1.0 / m)
                      - mean * mean, 0.0)
    scale = g_ref[...] * jax.lax.rsqrt(var + eps)
    shift = b_ref[...] - mean * scale
    o_ref[...] = jnp.maximum(y * scale + shift, 0.0).astype(o_ref.dtype)


def _nhwc_stage_kernel(p_ref, w_ref, g_ref, b_ref, o_ref, *, cout, eps):
    y = jnp.dot(p_ref[...], w_ref[...], preferred_element_type=jnp.float32)
    m = y.shape[0]
    s = jnp.sum(y, axis=0, keepdims=True)
    q = jnp.sum(y * y, axis=0, keepdims=True)
    sc = s[:, 0:cout] + s[:, cout:2 * cout] + s[:, 2 * cout:3 * cout] \
        + s[:, 3 * cout:4 * cout]
    qc = q[:, 0:cout] + q[:, cout:2 * cout] + q[:, 2 * cout:3 * cout] \
        + q[:, 3 * cout:4 * cout]
    cnt = 4.0 * m
    mean = sc * (1.0 / cnt)
    var = jnp.maximum(qc * (1.0 / cnt) - mean * mean, 0.0)
    scale = g_ref[...] * jax.lax.rsqrt(var + eps)
    shift = b_ref[...] - mean * scale
    scale4 = jnp.concatenate([scale, scale, scale, scale], axis=1)
    shift4 = jnp.concatenate([shift, shift, shift, shift], axis=1)
    o_ref[...] = jnp.maximum(y * scale4 + shift4, 0.0).astype(o_ref.dtype)


def _conv_stats_kernel(x_ref, w_ref, y_ref, st_ref, *, H, W):
    patches = _gather3x3(x_ref[0], H, W)
    y = jnp.dot(w_ref[...], patches, preferred_element_type=jnp.float32)
    y_ref[0] = y.astype(y_ref.dtype)
    s = jnp.sum(y, axis=1, keepdims=True)
    q = jnp.sum(y * y, axis=1, keepdims=True)
    st_ref[0] = jnp.concatenate([s, q], axis=1)


def _bn_conv_stats_kernel(x_ref, st_ref, g_ref, b_ref, w_ref, y_ref, so_ref,
                          *, H, W, cnt, eps):
    C = x_ref.shape[1]
    st = jnp.sum(st_ref[...], axis=0)
    stc = _fold_stats(st, C)
    a = _bn_apply(x_ref[0].astype(jnp.float32), stc, g_ref[...], b_ref[...],
                  cnt, eps).astype(jnp.bfloat16)
    patches = _gather3x3(a, H, W)
    y = jnp.dot(w_ref[...], patches, preferred_element_type=jnp.float32)
    y_ref[0] = y.astype(y_ref.dtype)
    s = jnp.sum(y, axis=1, keepdims=True)
    q = jnp.sum(y * y, axis=1, keepdims=True)
    so_ref[0] = jnp.concatenate([s, q], axis=1)


def _bn_conv_kernel(x_ref, st_ref, g_ref, b_ref, w_ref, y_ref,
                    *, H, W, cnt, eps):
    C = x_ref.shape[1]
    st = jnp.sum(st_ref[...], axis=0)
    stc = _fold_stats(st, C)
    a = _bn_apply(x_ref[0].astype(jnp.float32), stc, g_ref[...], b_ref[...],
                  cnt, eps).astype(jnp.bfloat16)
    patches = _gather3x3(a, H, W)
    y = jnp.dot(w_ref[...], patches, preferred_element_type=jnp.float32)
    y_ref[0] = y.astype(y_ref.dtype)


def _tail_kernel(x_ref, n1_ref, n2_ref, w1_ref, b1_ref, w2_ref, b2_ref,
                 w3_ref, b3_ref, w4_ref, b4_ref, o_ref, *, H, W, nc,
                 upper, lower):
    S = H * W
    a = x_ref[0].astype(jnp.float32)
    a = _gen_noise(a, n1_ref[0], upper, lower)
    a = _conv3x3(a, w1_ref[...], b1_ref[...], H, W)
    a = _conv3x3(a, w2_ref[...], b2_ref[...], H, W)
    n2 = jnp.concatenate(
        [n2_ref[0], jnp.zeros((a.shape[0] - n2_ref.shape[1], S), jnp.float32)],
        axis=0)
    a = _gen_noise(a, n2, upper, lower)
    a = _conv3x3(a, w3_ref[...], b3_ref[...], H, W)
    a = _conv3x3(a, w4_ref[...], b4_ref[...], H, W)
    o_ref[0] = jnp.tanh(a[:nc, :])


def _stage1(a1, w1, gamma, beta):
    M, K = a1.shape
    C = w1.shape[1]
    return pl.pallas_call(
        functools.partial(_stage1_kernel, eps=_EPS),
        out_shape=jax.ShapeDtypeStruct((M, C), jnp.bfloat16),
        grid=(1,),
        in_specs=[pl.BlockSpec((M, K), lambda i: (0, 0)),
                  pl.BlockSpec((K, C), lambda i: (0, 0)),
                  pl.BlockSpec((1, C), lambda i: (0, 0)),
                  pl.BlockSpec((1, C), lambda i: (0, 0))],
        out_specs=pl.BlockSpec((M, C), lambda i: (0, 0)),
        compiler_params=pltpu.CompilerParams(
            dimension_semantics=("arbitrary",)),
    )(a1.astype(jnp.bfloat16), w1, gamma, beta)


def _nhwc_stage(patches, wcomb, gamma, beta, *, cout):
    M, K = patches.shape
    Ncol = wcomb.shape[1]
    return pl.pallas_call(
        functools.partial(_nhwc_stage_kernel, cout=cout, eps=_EPS),
        out_shape=jax.ShapeDtypeStruct((M, Ncol), jnp.bfloat16),
        grid=(1,),
        in_specs=[pl.BlockSpec((M, K), lambda i: (0, 0)),
                  pl.BlockSpec((K, Ncol), lambda i: (0, 0)),
                  pl.BlockSpec((1, cout), lambda i: (0, 0)),
                  pl.BlockSpec((1, cout), lambda i: (0, 0))],
        out_specs=pl.BlockSpec((M, Ncol), lambda i: (0, 0)),
        compiler_params=pltpu.CompilerParams(
            dimension_semantics=("arbitrary",)),
    )(patches, wcomb, gamma, beta)


def _conv_stats(x, wpl, *, H, W):
    N, Cin, S = x.shape
    R, K = wpl.shape
    return pl.pallas_call(
        functools.partial(_conv_stats_kernel, H=H, W=W),
        out_shape=(jax.ShapeDtypeStruct((N, R, S), jnp.bfloat16),
                   jax.ShapeDtypeStruct((N, R, 2), jnp.float32)),
        grid=(N,),
        in_specs=[pl.BlockSpec((1, Cin, S), lambda n: (n, 0, 0)),
                  pl.BlockSpec((R, K), lambda n: (0, 0))],
        out_specs=(pl.BlockSpec((1, R, S), lambda n: (n, 0, 0)),
                   pl.BlockSpec((1, R, 2), lambda n: (n, 0, 0))),
        compiler_params=pltpu.CompilerParams(
            dimension_semantics=("parallel",),
            vmem_limit_bytes=_VMEM_LIMIT),
    )(x, wpl)


def _bn_conv_stats(x, st, gc, bc, wpl, *, H, W, cnt):
    N, Cin, S = x.shape
    R, K = wpl.shape
    Rp = st.shape[1]
    return pl.pallas_call(
        functools.partial(_bn_conv_stats_kernel, H=H, W=W, cnt=cnt, eps=_EPS),
        out_shape=(jax.ShapeDtypeStruct((N, R, S), jnp.bfloat16),
                   jax.ShapeDtypeStruct((N, R, 2), jnp.float32)),
        grid=(N,),
        in_specs=[pl.BlockSpec((1, Cin, S), lambda n: (n, 0, 0)),
                  pl.BlockSpec((N, Rp, 2), lambda n: (0, 0, 0)),
                  pl.BlockSpec((Cin, 1), lambda n: (0, 0)),
                  pl.BlockSpec((Cin, 1), lambda n: (0, 0)),
                  pl.BlockSpec((R, K), lambda n: (0, 0))],
        out_specs=(pl.BlockSpec((1, R, S), lambda n: (n, 0, 0)),
                   pl.BlockSpec((1, R, 2), lambda n: (n, 0, 0))),
        compiler_params=pltpu.CompilerParams(
            dimension_semantics=("parallel",),
            vmem_limit_bytes=_VMEM_LIMIT),
    )(x, st, gc, bc, wpl)


def _bn_conv(x, st, gc, bc, wpl, *, H, W, cnt):
    N, Cin, S = x.shape
    R, K = wpl.shape
    Rp = st.shape[1]
    return pl.pallas_call(
        functools.partial(_bn_conv_kernel, H=H, W=W, cnt=cnt, eps=_EPS),
        out_shape=jax.ShapeDtypeStruct((N, R, S), jnp.bfloat16),
        grid=(N,),
        in_specs=[pl.BlockSpec((1, Cin, S), lambda n: (n, 0, 0)),
                  pl.BlockSpec((N, Rp, 2), lambda n: (0, 0, 0)),
                  pl.BlockSpec((Cin, 1), lambda n: (0, 0)),
                  pl.BlockSpec((Cin, 1), lambda n: (0, 0)),
                  pl.BlockSpec((R, K), lambda n: (0, 0))],
        out_specs=pl.BlockSpec((1, R, S), lambda n: (n, 0, 0)),
        compiler_params=pltpu.CompilerParams(
            dimension_semantics=("parallel",),
            vmem_limit_bytes=_VMEM_LIMIT),
    )(x, st, gc, bc, wpl)


def _tail(act, n1, n2, weights, *, nc, H, W, upper=4.0, lower=2.0):
    N, C0, S = act.shape
    c2 = n2.shape[1]
    w1, b1, w2, b2, w3, b3, w4, b4 = weights

    def rep_spec(arr):
        nd = arr.ndim
        return pl.BlockSpec(arr.shape, lambda n, nd=nd: (0,) * nd)

    return pl.pallas_call(
        functools.partial(_tail_kernel, H=H, W=W, nc=nc,
                          upper=upper, lower=lower),
        out_shape=jax.ShapeDtypeStruct((N, nc, S), jnp.float32),
        grid=(N,),
        in_specs=[pl.BlockSpec((1, C0, S), lambda n: (n, 0, 0)),
                  pl.BlockSpec((1, C0, S), lambda n: (n, 0, 0)),
                  pl.BlockSpec((1, c2, S), lambda n: (n, 0, 0)),
                  rep_spec(w1), rep_spec(b1), rep_spec(w2), rep_spec(b2),
                  rep_spec(w3), rep_spec(b3), rep_spec(w4), rep_spec(b4)],
        out_specs=pl.BlockSpec((1, nc, S), lambda n: (n, 0, 0)),
        compiler_params=pltpu.CompilerParams(
            dimension_semantics=("parallel",),
            vmem_limit_bytes=_VMEM_LIMIT),
    )(act, n1, n2, w1, b1, w2, b2, w3, b3, w4, b4)


def _nhwc_patches(x_nhwc):
    N, H, W, C = x_nhwc.shape
    xp = jnp.pad(x_nhwc, ((0, 0), (1, 1), (1, 1), (0, 0)))
    cols = [xp[:, wy:wy + H, wx:wx + W, :]
            for wy in range(3) for wx in range(3)]
    return jnp.stack(cols, axis=3).reshape(N * H * W, 9 * C)


def _nhwc_uninterleave(z, N, H, W, C):
    z = z.reshape(N, H, W, 2, 2, C).transpose(0, 1, 3, 2, 4, 5)
    return z.reshape(N, 2 * H, 2 * W, C)


def _planar_uninterleave(y, N, C, H, W):
    img = y.reshape(N, C, 2, 2, H, W).transpose(0, 1, 4, 2, 5, 3)
    return img.reshape(N, C, 4 * H * W)


def kernel(m1, m2, m3, m4, m5, m6,
           g1, b1, g2, b2, g3, b3, g4, b4, g5, b5,
           c1_w, c1_b, c2_w, c2_b, c3_w, c3_b, c4_w, c4_b,
           x, noise1, noise2):
    nc, ngf = 1, 16
    N, nz = x.shape[0], x.shape[1]
    z = x.reshape(N, nz).astype(jnp.bfloat16)

    eye16 = jnp.eye(16, dtype=z.dtype)
    a1 = (eye16[None, :, :, None] * z[:, None, None, :]).reshape(
        N * 16, 16 * nz)
    h = _stage1(a1, m1, g1, b1)
    h = h.reshape(N, 4, 4, ngf * 16)

    h = _nhwc_uninterleave(
        _nhwc_stage(_nhwc_patches(h), m2, g2, b2, cout=ngf * 8),
        N, 4, 4, ngf * 8)
    h = _nhwc_uninterleave(
        _nhwc_stage(_nhwc_patches(h), m3, g3, b3, cout=ngf * 4),
        N, 8, 8, ngf * 4)
    hp = jnp.transpose(h, (0, 3, 1, 2)).reshape(N, ngf * 4, 256)

    y4, st4 = _conv_stats(hp, m4, H=16, W=16)
    y4u = _planar_uninterleave(y4, N, ngf * 2, 16, 16)

    y5, st5 = _bn_conv_stats(y4u, st4, g4[::4], b4[::4], m5,
                             H=32, W=32, cnt=float(N * 1024))
    y5u = _planar_uninterleave(y5, N, ngf, 32, 32)

    y6 = _bn_conv(y5u, st5, g5[::4], b5[::4], m6,
                  H=64, W=64, cnt=float(N * 4096))
    act = _planar_uninterleave(y6, N, ngf // 2, 64, 64)

    S = 128 * 128
    n1 = noise1.reshape(N, ngf // 2, S)
    n2 = noise2.reshape(N, noise2.shape[1], S)
    weights = (c1_w, c1_b, c2_w, c2_b, c3_w, c3_b, c4_w, c4_b)
    out = _tail(act, n1, n2, weights, nc=nc, H=128, W=128)
    return out.reshape(N, nc, 128, 128)


# speedup vs baseline: 1.0510x; 1.0510x over previous
import functools

import jax
import jax.numpy as jnp
from jax.experimental import pallas as pl
from jax.experimental.pallas import tpu as pltpu

_EPS = 1e-5
_VMEM_LIMIT = 48 * 1024 * 1024


def _lane_shift(x, d):
    S = x.shape[-1]
    d = d % S
    if d == 0:
        return x
    return jnp.concatenate([x[:, d:], x[:, :d]], axis=-1)


def _gather3x3(x, H, W):
    S = H * W
    col = jax.lax.broadcasted_iota(jnp.int32, (1, S), 1)
    yy = col // W
    xx = col % W
    parts = []
    for wy in range(3):
        for wx in range(3):
            dy, dx = wy - 1, wx - 1
            shifted = _lane_shift(x, dy * W + dx)
            valid = ((yy + dy >= 0) & (yy + dy < H) &
                     (xx + dx >= 0) & (xx + dx < W))
            parts.append(jnp.where(valid, shifted, 0.0))
    return jnp.concatenate(parts, axis=0)


def _gen_noise(x, noise, upper, lower):
    S = x.shape[-1]
    cmax = jnp.max(x, axis=-1, keepdims=True)
    s = jnp.sum(x, axis=-1, keepdims=True)
    q = jnp.sum(x * x, axis=-1, keepdims=True)
    mean = s * (1.0 / S)
    var = jnp.maximum((q - S * mean * mean) * (1.0 / (S - 1)), 0.0)
    std = jnp.sqrt(var)
    clone = jnp.where(x < -cmax * (1.0 / lower), 0.0, x)
    clone = jnp.where(clone > cmax * (1.0 / upper), 0.0, clone)
    return x + clone * (noise * std)


def _conv3x3_trim(a, wmat, bias, H, W, cin):
    patches = _gather3x3(a[:cin].astype(jnp.bfloat16), H, W)
    y = jnp.dot(wmat, patches, preferred_element_type=jnp.float32)
    return y + bias


def _ct1_bn_kernel(a_ref, w_ref, g_ref, b_ref, o_ref, *, eps):
    y = jnp.dot(a_ref[...], w_ref[...], preferred_element_type=jnp.float32)
    m = y.shape[0]
    mean = jnp.sum(y, axis=0, keepdims=True) * (1.0 / m)
    var = jnp.maximum(jnp.sum(y * y, axis=0, keepdims=True) * (1.0 / m)
                      - mean * mean, 0.0)
    scale = g_ref[...] * jax.lax.rsqrt(var + eps)
    shift = b_ref[...] - mean * scale
    o_ref[...] = jnp.maximum(y * scale + shift, 0.0).astype(o_ref.dtype)


def _nhwc_ct_bn_kernel(p_ref, w_ref, g_ref, b_ref, o_ref, *, cout, eps):
    y = jnp.dot(p_ref[...], w_ref[...], preferred_element_type=jnp.float32)
    m = y.shape[0]
    s = jnp.sum(y, axis=0, keepdims=True)
    q = jnp.sum(y * y, axis=0, keepdims=True)
    sc = s[:, 0:cout] + s[:, cout:2 * cout] + s[:, 2 * cout:3 * cout] \
        + s[:, 3 * cout:4 * cout]
    qc = q[:, 0:cout] + q[:, cout:2 * cout] + q[:, 2 * cout:3 * cout] \
        + q[:, 3 * cout:4 * cout]
    cnt = 4.0 * m
    mean = sc * (1.0 / cnt)
    var = jnp.maximum(qc * (1.0 / cnt) - mean * mean, 0.0)
    scale = g_ref[...] * jax.lax.rsqrt(var + eps)
    shift = b_ref[...] - mean * scale
    scale4 = jnp.concatenate([scale, scale, scale, scale], axis=1)
    shift4 = jnp.concatenate([shift, shift, shift, shift], axis=1)
    o_ref[...] = jnp.maximum(y * scale4 + shift4, 0.0).astype(o_ref.dtype)


def _planar_ct_bn_kernel(x_ref, w_ref, g_ref, b_ref, o_ref, *, H, W, eps):
    n_batch = x_ref.shape[0]
    S = H * W
    w = w_ref[...]
    ys, s_acc, q_acc = [], None, None
    for n in range(n_batch):
        patches = _gather3x3(x_ref[n], H, W)
        y = jnp.dot(w, patches, preferred_element_type=jnp.float32)
        ys.append(y)
        s_n = jnp.sum(y, axis=1, keepdims=True)
        q_n = jnp.sum(y * y, axis=1, keepdims=True)
        s_acc = s_n if s_acc is None else s_acc + s_n
        q_acc = q_n if q_acc is None else q_acc + q_n
    r = s_acc.shape[0]
    ri = jax.lax.broadcasted_iota(jnp.int32, (r, r), 0) // 4
    cj = jax.lax.broadcasted_iota(jnp.int32, (r, r), 1) // 4
    fold = (ri == cj).astype(jnp.float32)
    stats = jnp.dot(fold, jnp.concatenate([s_acc, q_acc], axis=1),
                    preferred_element_type=jnp.float32)
    cnt = float(n_batch * 4 * S)
    mean = stats[:, 0:1] * (1.0 / cnt)
    var = jnp.maximum(stats[:, 1:2] * (1.0 / cnt) - mean * mean, 0.0)
    scale = g_ref[...] * jax.lax.rsqrt(var + eps)
    shift = b_ref[...] - mean * scale
    for n in range(n_batch):
        o_ref[n] = jnp.maximum(ys[n] * scale + shift, 0.0).astype(o_ref.dtype)


def _ct6_kernel(x_ref, w_ref, o_ref, *, H, W):
    patches = _gather3x3(x_ref[0], H, W)
    y = jnp.dot(w_ref[...], patches, preferred_element_type=jnp.float32)
    o_ref[0] = y.astype(o_ref.dtype)


def _tail_kernel(x_ref, n1_ref, n2_ref, w1_ref, b1_ref, w2_ref, b2_ref,
                 w3_ref, b3_ref, w4_ref, b4_ref, o_ref, *, H, W, nc,
                 upper, lower):
    S = H * W
    nb = x_ref.shape[0]
    c2 = n2_ref.shape[1]
    for i in range(nb):
        a = x_ref[i].astype(jnp.float32)
        a = _gen_noise(a, n1_ref[i], upper, lower)
        a = _conv3x3_trim(a, w1_ref[...], b1_ref[...], H, W, 8)
        a = _conv3x3_trim(a, w2_ref[...], b2_ref[...], H, W, 4)
        n2 = jnp.concatenate(
            [n2_ref[i], jnp.zeros((a.shape[0] - c2, S), jnp.float32)], axis=0)
        a = _gen_noise(a, n2, upper, lower)
        a = _conv3x3_trim(a, w3_ref[...], b3_ref[...], H, W, 2)
        a = _conv3x3_trim(a, w4_ref[...], b4_ref[...], H, W, 1)
        o_ref[i] = jnp.tanh(a[:nc, :])


def _ct1_bn_relu(a1, w1, gamma, beta):
    M, K = a1.shape
    C = w1.shape[1]
    return pl.pallas_call(
        functools.partial(_ct1_bn_kernel, eps=_EPS),
        out_shape=jax.ShapeDtypeStruct((M, C), jnp.bfloat16),
        grid=(1,),
        in_specs=[pl.BlockSpec((M, K), lambda i: (0, 0)),
                  pl.BlockSpec((K, C), lambda i: (0, 0)),
                  pl.BlockSpec((1, C), lambda i: (0, 0)),
                  pl.BlockSpec((1, C), lambda i: (0, 0))],
        out_specs=pl.BlockSpec((M, C), lambda i: (0, 0)),
        compiler_params=pltpu.CompilerParams(
            dimension_semantics=("arbitrary",)),
    )(a1.astype(jnp.bfloat16), w1, gamma, beta)


def _nhwc_ct_bn_relu(patches, wcomb, gamma, beta, *, cout):
    M, K = patches.shape
    Ncol = wcomb.shape[1]
    return pl.pallas_call(
        functools.partial(_nhwc_ct_bn_kernel, cout=cout, eps=_EPS),
        out_shape=jax.ShapeDtypeStruct((M, Ncol), jnp.bfloat16),
        grid=(1,),
        in_specs=[pl.BlockSpec((M, K), lambda i: (0, 0)),
                  pl.BlockSpec((K, Ncol), lambda i: (0, 0)),
                  pl.BlockSpec((1, cout), lambda i: (0, 0)),
                  pl.BlockSpec((1, cout), lambda i: (0, 0))],
        out_specs=pl.BlockSpec((M, Ncol), lambda i: (0, 0)),
        compiler_params=pltpu.CompilerParams(
            dimension_semantics=("arbitrary",)),
    )(patches, wcomb, gamma, beta)


def _planar_ct_bn_relu(x, wpl, gamma_rows, beta_rows, *, H, W):
    N, Cin, S = x.shape
    R, K = wpl.shape
    return pl.pallas_call(
        functools.partial(_planar_ct_bn_kernel, H=H, W=W, eps=_EPS),
        out_shape=jax.ShapeDtypeStruct((N, R, S), jnp.bfloat16),
        grid=(1,),
        in_specs=[pl.BlockSpec((N, Cin, S), lambda i: (0, 0, 0)),
                  pl.BlockSpec((R, K), lambda i: (0, 0)),
                  pl.BlockSpec((R, 1), lambda i: (0, 0)),
                  pl.BlockSpec((R, 1), lambda i: (0, 0))],
        out_specs=pl.BlockSpec((N, R, S), lambda i: (0, 0, 0)),
        compiler_params=pltpu.CompilerParams(
            dimension_semantics=("arbitrary",),
            vmem_limit_bytes=_VMEM_LIMIT),
    )(x, wpl, gamma_rows, beta_rows)


def _ct6_phase(x, wpl, *, H, W):
    N, Cin, S = x.shape
    R, K = wpl.shape
    return pl.pallas_call(
        functools.partial(_ct6_kernel, H=H, W=W),
        out_shape=jax.ShapeDtypeStruct((N, R, S), jnp.bfloat16),
        grid=(N,),
        in_specs=[pl.BlockSpec((1, Cin, S), lambda n: (n, 0, 0)),
                  pl.BlockSpec((R, K), lambda n: (0, 0))],
        out_specs=pl.BlockSpec((1, R, S), lambda n: (n, 0, 0)),
        compiler_params=pltpu.CompilerParams(
            dimension_semantics=("parallel",),
            vmem_limit_bytes=_VMEM_LIMIT),
    )(x, wpl)


def _tail(act, n1, n2, weights, *, nc, H, W, nb=2, upper=4.0, lower=2.0):
    N, C0, S = act.shape
    c2 = n2.shape[1]
    w1, b1, w2, b2, w3, b3, w4, b4 = weights

    def rep_spec(arr):
        nd = arr.ndim
        return pl.BlockSpec(arr.shape, lambda n, nd=nd: (0,) * nd)

    return pl.pallas_call(
        functools.partial(_tail_kernel, H=H, W=W, nc=nc,
                          upper=upper, lower=lower),
        out_shape=jax.ShapeDtypeStruct((N, nc, S), jnp.float32),
        grid=(N // nb,),
        in_specs=[pl.BlockSpec((nb, C0, S), lambda n: (n, 0, 0)),
                  pl.BlockSpec((nb, C0, S), lambda n: (n, 0, 0)),
                  pl.BlockSpec((nb, c2, S), lambda n: (n, 0, 0)),
                  rep_spec(w1), rep_spec(b1), rep_spec(w2), rep_spec(b2),
                  rep_spec(w3), rep_spec(b3), rep_spec(w4), rep_spec(b4)],
        out_specs=pl.BlockSpec((nb, nc, S), lambda n: (n, 0, 0)),
        compiler_params=pltpu.CompilerParams(
            dimension_semantics=("parallel",),
            vmem_limit_bytes=_VMEM_LIMIT),
    )(act, n1, n2, w1, b1, w2, b2, w3, b3, w4, b4)


def _nhwc_patches(x_nhwc):
    N, H, W, C = x_nhwc.shape
    xp = jnp.pad(x_nhwc, ((0, 0), (1, 1), (1, 1), (0, 0)))
    cols = [xp[:, wy:wy + H, wx:wx + W, :]
            for wy in range(3) for wx in range(3)]
    return jnp.stack(cols, axis=3).reshape(N * H * W, 9 * C)


def _nhwc_uninterleave(z, N, H, W, C):
    z = z.reshape(N, H, W, 2, 2, C).transpose(0, 1, 3, 2, 4, 5)
    return z.reshape(N, 2 * H, 2 * W, C)


def _planar_uninterleave(y, N, C, H, W):
    img = y.reshape(N, C, 2, 2, H, W).transpose(0, 1, 4, 2, 5, 3)
    return img.reshape(N, C, 4 * H * W)


def _trim_conv_w(w, cin):
    return w.reshape(8, 9, 8)[:, :, :cin].reshape(8, 9 * cin)


def kernel(m1, m2, m3, m4, m5, m6,
           g1, b1, g2, b2, g3, b3, g4, b4, g5, b5,
           c1_w, c1_b, c2_w, c2_b, c3_w, c3_b, c4_w, c4_b,
           x, noise1, noise2):
    nc, ngf = 1, 16
    N, nz = x.shape[0], x.shape[1]
    z = x.reshape(N, nz).astype(jnp.bfloat16)

    eye16 = jnp.eye(16, dtype=z.dtype)
    a1 = (eye16[None, :, :, None] * z[:, None, None, :]).reshape(
        N * 16, 16 * nz)
    h = _ct1_bn_relu(a1, m1, g1, b1)
    h = h.reshape(N, 4, 4, ngf * 16)

    h = _nhwc_uninterleave(
        _nhwc_ct_bn_relu(_nhwc_patches(h), m2, g2, b2, cout=ngf * 8),
        N, 4, 4, ngf * 8)
    h = _nhwc_uninterleave(
        _nhwc_ct_bn_relu(_nhwc_patches(h), m3, g3, b3, cout=ngf * 4),
        N, 8, 8, ngf * 4)
    hp = jnp.transpose(h, (0, 3, 1, 2)).reshape(N, ngf * 4, 256)

    y = _planar_ct_bn_relu(hp, m4, g4, b4, H=16, W=16)
    hp = _planar_uninterleave(y, N, ngf * 2, 16, 16)
    y = _planar_ct_bn_relu(hp, m5, g5, b5, H=32, W=32)
    hp = _planar_uninterleave(y, N, ngf, 32, 32)

    y = _ct6_phase(hp, m6, H=64, W=64)
    act = _planar_uninterleave(y, N, ngf // 2, 64, 64)

    S = 128 * 128
    n1 = noise1.reshape(N, ngf // 2, S)
    n2 = noise2.reshape(N, noise2.shape[1], S)
    weights = (c1_w, c1_b,
               _trim_conv_w(c2_w, 4), c2_b,
               _trim_conv_w(c3_w, 2), c3_b,
               _trim_conv_w(c4_w, 1), c4_b)
    out = _tail(act, n1, n2, weights, nc=nc, H=128, W=128)
    return out.reshape(N, nc, 128, 128)
